# baseline (speedup 1.0000x reference)
import jax
import jax.numpy as jnp
from jax import lax
from jax.experimental import pallas as pl
from jax.experimental.pallas import tpu as pltpu
from jax.experimental.pallas import tpu_sc as plsc

_VOCAB = 1000000
_HID = 64
_PAD = 1
_B = 1024
_L = 200
_EPS = 1e-12

_NC = 2
_NS = 16
_NW = _NC * _NS
_TOK = _B * _L
_TPW = _TOK // _NW
_RPW = _B // _NW
_CH = 128
_NCH = _TPW // _CH
_NPOS = 208


def _rsqrt16(x):
    xi = plsc.bitcast(x, jnp.int32)
    yi = 0x5F3759DF - lax.shift_right_logical(xi, 1)
    y = plsc.bitcast(yi, jnp.float32)
    half_x = 0.5 * x
    for _ in range(4):
        y = y * (1.5 - half_x * y * y)
    return y


def _sc_body(ids_hbm, tt_hbm, it_hbm, word_hbm, pos_hbm, type_hbm, item_hbm,
             g_hbm, b_hbm, out_hbm,
             ids_v, widx_v, tt_v, it_v, pid_v, pos2_v, pos_v, type_v, item_v,
             g_v, b_v, w0, w1, ev, sb, qb, mb, ib, out_v,
             sem_w0, sem_w1):
    i32 = jnp.int32
    wid = lax.axis_index("s") * _NC + lax.axis_index("c")
    base = wid * _TPW

    pltpu.sync_copy(ids_hbm.at[pl.ds(base, _TPW)], ids_v)
    pltpu.sync_copy(tt_hbm.at[pl.ds(base, _TPW)], tt_v)
    pltpu.sync_copy(it_hbm.at[pl.ds(base, _TPW)], it_v)
    pltpu.sync_copy(pos_hbm.at[pl.ds(0, _NPOS // 2)], pos2_v)
    pltpu.sync_copy(type_hbm, type_v)
    pltpu.sync_copy(item_hbm, item_v)
    pltpu.sync_copy(g_hbm, g_v)
    pltpu.sync_copy(b_hbm, b_v)

    lanes = lax.iota(i32, 16)

    def widx_body(i):
        v = ids_v[pl.ds(i * 16, 16)]
        widx_v[pl.ds(i * 16, 16)] = lax.shift_right_logical(v, 1)
    plsc.parallel_loop(0, _TPW // 16, unroll=8)(widx_body)

    def pos_body(i):
        r = i // 8
        c = i % 8
        pos_v[pl.ds(i * 16, 16)] = pos2_v[r, pl.ds(c * 16, 16)]
    plsc.parallel_loop(0, _NPOS * 4, unroll=8)(pos_body)

    for g2 in range(_RPW // 16):
        row_off = lanes * _L + g2 * 16 * _L

        def cum_body(l, cum, row_off=row_off):
            idv = plsc.load_gather(ids_v, [row_off + l])
            m = jnp.where(idv != _PAD, 1, 0).astype(i32)
            cum = cum + m
            plsc.store_scatter(pid_v, [row_off + l], cum * m + _PAD)
            return cum
        plsc.parallel_loop(0, _L, unroll=8,
                           carry=jnp.zeros((16,), i32))(cum_body)

    gvecs = [g_v[pl.ds(j * 16, 16)] for j in range(4)]
    bvecs = [b_v[pl.ds(j * 16, 16)] for j in range(4)]

    def issue(c, wb, sw):
        pltpu.async_copy(word_hbm.at[widx_v.at[pl.ds(c * _CH, _CH)]], wb, sw)

    def compute(c, wb):
        tok0 = c * _CH

        def p1_body(g):
            idsv = ids_v[pl.ds(tok0 + g * 16, 16)]
            parv = (idsv & 1) * _HID
            pidv = pid_v[pl.ds(tok0 + g * 16, 16)] * _HID
            ttv = tt_v[pl.ds(tok0 + g * 16, 16)] * _HID
            itv = it_v[pl.ds(tok0 + g * 16, 16)] * _HID
            for k in range(16):
                t = g * 16 + k
                par = parv[k]
                pp = pidv[k]
                tt = ttv[k]
                it = itv[k]
                e = [wb[t, pl.ds(par + j * 16, 16)]
                     + pos_v[pl.ds(pp + j * 16, 16)]
                     + type_v[pl.ds(tt + j * 16, 16)]
                     + item_v[pl.ds(it + j * 16, 16)]
                     for j in range(4)]
                s = (e[0] + e[1]) + (e[2] + e[3])
                q = ((e[0] * e[0] + e[1] * e[1])
                     + (e[2] * e[2] + e[3] * e[3]))
                for j in range(4):
                    ev[pl.ds(t * _HID + j * 16, 16)] = e[j]
                sb[pl.ds(t * 24, 16)] = plsc.cumsum(s)
                qb[pl.ds(t * 24, 16)] = plsc.cumsum(q)
        plsc.parallel_loop(0, _CH // 16, unroll=2)(p1_body)

        lane24 = lanes * 24 + 15
        def p2_body(g):
            sumv = plsc.load_gather(sb, [lane24 + g * (16 * 24)])
            sqv = plsc.load_gather(qb, [lane24 + g * (16 * 24)])
            mean = sumv * (1.0 / _HID)
            var = sqv * (1.0 / _HID) - mean * mean
            mb[pl.ds(g * 16, 16)] = mean
            ib[pl.ds(g * 16, 16)] = _rsqrt16(var + _EPS)
        plsc.parallel_loop(0, _CH // 16, unroll=2)(p2_body)

        def p3_body(g):
            mv = mb[pl.ds(g * 16, 16)]
            iv = ib[pl.ds(g * 16, 16)]
            for k in range(16):
                t = g * 16 + k
                mean = jnp.full((16,), mv[k], jnp.float32)
                inv = jnp.full((16,), iv[k], jnp.float32)
                for j in range(4):
                    ej = ev[pl.ds(t * _HID + j * 16, 16)]
                    out_v[pl.ds(t * _HID + j * 16, 16)] = (
                        (ej - mean) * inv * gvecs[j] + bvecs[j])
        plsc.parallel_loop(0, _CH // 16, unroll=2)(p3_body)

        pltpu.sync_copy(out_v, out_hbm.at[pl.ds((base + tok0) * _HID,
                                                _CH * _HID)])

    issue(0, w0, sem_w0)

    def drain(buf, sem):
        pltpu.make_async_copy(word_hbm.at[pl.ds(0, _CH)], buf,
                              sem).wait()

    def body2(c2, _):
        c = c2 * 2
        drain(w0, sem_w0)
        issue(c + 1, w1, sem_w1)
        compute(c, w0)
        drain(w1, sem_w1)
        nxt = jnp.minimum(c + 2, _NCH - 2)
        issue(nxt, w0, sem_w0)
        compute(c + 1, w1)
        return 0

    lax.fori_loop(0, _NCH // 2, body2, 0)
    drain(w0, sem_w0)


@jax.jit
def kernel(input_ids, token_type_ids, item_position_ids, word_emb, pos_emb,
           type_emb, item_emb, ln_gamma, ln_beta):
    ids = input_ids.reshape(-1).astype(jnp.int32)
    tt = token_type_ids.reshape(-1).astype(jnp.int32)
    it = item_position_ids.reshape(-1).astype(jnp.int32)
    word2 = word_emb.reshape(_VOCAB // 2, 2 * _HID)
    pos2 = pos_emb.reshape(-1, 2 * _HID)

    k = pl.kernel(
        _sc_body,
        mesh=plsc.VectorSubcoreMesh(core_axis_name="c", subcore_axis_name="s"),
        compiler_params=pltpu.CompilerParams(needs_layout_passes=False,
                                             use_tc_tiling_on_sc=True),
        out_type=jax.ShapeDtypeStruct((_TOK * _HID,), jnp.float32),
        scratch_types=[
            pltpu.VMEM((_TPW,), jnp.int32),
            pltpu.VMEM((_TPW,), jnp.int32),
            pltpu.VMEM((_TPW,), jnp.int32),
            pltpu.VMEM((_TPW,), jnp.int32),
            pltpu.VMEM((_TPW,), jnp.int32),
            pltpu.VMEM((_NPOS // 2, 128), jnp.float32),
            pltpu.VMEM((_NPOS * _HID,), jnp.float32),
            pltpu.VMEM((4 * _HID,), jnp.float32),
            pltpu.VMEM((32 * _HID,), jnp.float32),
            pltpu.VMEM((_HID,), jnp.float32),
            pltpu.VMEM((_HID,), jnp.float32),
            pltpu.VMEM((_CH, 128), jnp.float32),
            pltpu.VMEM((_CH, 128), jnp.float32),
            pltpu.VMEM((_CH * _HID,), jnp.float32),
            pltpu.VMEM((_CH * 24,), jnp.float32),
            pltpu.VMEM((_CH * 24,), jnp.float32),
            pltpu.VMEM((_CH,), jnp.float32),
            pltpu.VMEM((_CH,), jnp.float32),
            pltpu.VMEM((_CH * _HID,), jnp.float32),
            pltpu.SemaphoreType.DMA,
            pltpu.SemaphoreType.DMA,
        ],
    )
    flat = k(ids, tt, it, word2, pos2, type_emb.reshape(-1),
             item_emb.reshape(-1), ln_gamma, ln_beta)
    return flat.reshape(_B, _L, _HID)
